# baseline (speedup 1.0000x reference)
import functools

import jax
import jax.numpy as jnp
from jax import lax
from jax.experimental import pallas as pl
from jax.experimental.pallas import tpu as pltpu
from jax.experimental.pallas import tpu_sc as plsc

N_SOURCES = 4096
TOP_K = 64
BATCH = 16384

_NC = 2
_NS = 16
_NW = _NC * _NS
_ROWS_PER_W = BATCH // _NW
_CHUNK = 128
_NCHUNKS = _ROWS_PER_W // _CHUNK


def _topk_body(imp_ref, ew_ref, rl_ref, idx_ref, c_ref):
    imp = imp_ref[:]
    gidx = (lax.broadcasted_iota(jnp.int32, (32, 128), 0) * 128
            + lax.broadcasted_iota(jnp.int32, (32, 128), 1))
    big = jnp.int32(1 << 30)
    neg = jnp.float32(-1e37)

    def sel_step(_, carry):
        impm, sel = carry
        m = jnp.max(impm)
        ij = jnp.min(jnp.where(impm == m, gidx, big))
        hit = gidx == ij
        return jnp.where(hit, neg, impm), sel | hit.astype(jnp.int32)

    _, sel = lax.fori_loop(
        0, TOP_K, sel_step, (imp, jnp.zeros((32, 128), jnp.int32)))

    lane = lax.broadcasted_iota(jnp.int32, (8, 128), 1)
    ew = ew_ref[:]

    def asc_step(k, carry):
        sel, idxv, ewv = carry
        ik = jnp.min(jnp.where(sel > 0, gidx, big))
        hit = gidx == ik
        ewk = jnp.sum(jnp.where(hit, ew, 0.0))
        put = lane == k
        return (sel * (1 - hit.astype(jnp.int32)),
                jnp.where(put, ik, idxv),
                jnp.where(put, ewk, ewv))

    _, idxv, ewv = lax.fori_loop(
        0, TOP_K, asc_step,
        (sel, jnp.zeros((8, 128), jnp.int32), jnp.zeros((8, 128), jnp.float32)))

    rl = rl_ref[:]
    m = jnp.max(rl, axis=1, keepdims=True)
    e = jnp.exp(rl - m)
    probs = e / jnp.sum(e, axis=1, keepdims=True)
    idx_ref[:] = idxv
    c_ref[:] = probs * ewv


def _topk_call(imp2, ew2, rl_pad):
    return pl.pallas_call(
        _topk_body,
        out_shape=[
            jax.ShapeDtypeStruct((8, 128), jnp.int32),
            jax.ShapeDtypeStruct((8, 128), jnp.float32),
        ],
    )(imp2, ew2, rl_pad)


def _route_body(src_hbm, idx_hbm, c0_hbm, c1_hbm, out0_hbm, out1_hbm,
                idx_v, c0_v, c1_v, idxl_a, idxl_b, data_a, data_b,
                out0_v, out1_v, sem_s, sem_a, sem_b):
    wid = lax.axis_index("s") * _NC + lax.axis_index("c")
    base = wid * _ROWS_PER_W

    pltpu.async_copy(idx_hbm, idx_v, sem_s).wait()
    pltpu.async_copy(c0_hbm, c0_v, sem_s).wait()
    pltpu.async_copy(c1_hbm, c1_v, sem_s).wait()

    idx_chunks = [idx_v[pl.ds(j * 16, 16)] for j in range(4)]
    c0_chunks = [c0_v[pl.ds(j * 16, 16)] for j in range(4)]
    c1_chunks = [c1_v[pl.ds(j * 16, 16)] for j in range(4)]

    def build(t, idxl):
        row0 = (base + t * _CHUNK) * N_SOURCES

        def body(r, _):
            off = row0 + r * N_SOURCES
            for j in range(4):
                idxl[pl.ds(r * 64 + j * 16, 16)] = idx_chunks[j] + off
            return 0

        lax.fori_loop(0, _CHUNK, body, 0)

    lane16 = lax.iota(jnp.int32, 16)
    _gdn = lax.GatherDimensionNumbers(
        offset_dims=(), collapsed_slice_dims=(0,), start_index_map=(0,))

    def _rotsum(v):
        for sh in (8, 4, 2, 1):
            ridx = (lane16 + sh) & 15
            v = v + lax.gather(v, ridx[:, None], _gdn, (1,),
                               mode=lax.GatherScatterMode.PROMISE_IN_BOUNDS)
        return v

    def compute(t, data):
        def body(g, _):
            vec0 = jnp.zeros((16,), jnp.float32)
            vec1 = jnp.zeros((16,), jnp.float32)
            for rr in range(16):
                off = g * (16 * 64) + rr * 64
                acc0 = data[pl.ds(off, 16)] * c0_chunks[0]
                acc1 = data[pl.ds(off, 16)] * c1_chunks[0]
                for j in range(1, 4):
                    d = data[pl.ds(off + j * 16, 16)]
                    acc0 = acc0 + d * c0_chunks[j]
                    acc1 = acc1 + d * c1_chunks[j]
                put = lane16 == rr
                vec0 = jnp.where(put, _rotsum(acc0), vec0)
                vec1 = jnp.where(put, _rotsum(acc1), vec1)
            out0_v[pl.ds(t * _CHUNK + g * 16, 16)] = vec0
            out1_v[pl.ds(t * _CHUNK + g * 16, 16)] = vec1
            return 0

        lax.fori_loop(0, _CHUNK // 16, body, 0)

    bufs = [(idxl_a, data_a, sem_a), (idxl_b, data_b, sem_b)]

    build(0, bufs[0][0])
    copies = {0: pltpu.async_copy(src_hbm.at[bufs[0][0]], bufs[0][1], bufs[0][2])}
    for t in range(_NCHUNKS):
        if t + 1 < _NCHUNKS:
            nb = bufs[(t + 1) % 2]
            build(t + 1, nb[0])
            copies[t + 1] = pltpu.async_copy(src_hbm.at[nb[0]], nb[1], nb[2])
        copies[t].wait()
        compute(t, bufs[t % 2][1])

    pltpu.async_copy(out0_v, out0_hbm.at[pl.ds(base, _ROWS_PER_W)], sem_s).wait()
    pltpu.async_copy(out1_v, out1_hbm.at[pl.ds(base, _ROWS_PER_W)], sem_s).wait()


@functools.partial(jax.jit, static_argnums=())
def _route_call(src_flat, idx64, c0, c1):
    mesh = plsc.VectorSubcoreMesh(core_axis_name="c", subcore_axis_name="s")
    f = pl.kernel(
        _route_body,
        out_type=[
            jax.ShapeDtypeStruct((BATCH,), jnp.float32),
            jax.ShapeDtypeStruct((BATCH,), jnp.float32),
        ],
        mesh=mesh,
        scratch_types=[
            pltpu.VMEM((TOP_K,), jnp.int32),
            pltpu.VMEM((TOP_K,), jnp.float32),
            pltpu.VMEM((TOP_K,), jnp.float32),
            pltpu.VMEM((_CHUNK * 64,), jnp.int32),
            pltpu.VMEM((_CHUNK * 64,), jnp.int32),
            pltpu.VMEM((_CHUNK * 64,), jnp.float32),
            pltpu.VMEM((_CHUNK * 64,), jnp.float32),
            pltpu.VMEM((_ROWS_PER_W,), jnp.float32),
            pltpu.VMEM((_ROWS_PER_W,), jnp.float32),
            pltpu.SemaphoreType.DMA,
            pltpu.SemaphoreType.DMA,
            pltpu.SemaphoreType.DMA,
        ],
    )
    return f(src_flat, idx64, c0, c1)


def kernel(sources, importance_logits, edge_weights, route_logits):
    imp2 = importance_logits.reshape(32, 128)
    ew2 = edge_weights.reshape(32, 128)
    rl_pad = jnp.full((8, 128), -1e30, jnp.float32)
    rl_pad = rl_pad.at[:2, :TOP_K].set(route_logits.astype(jnp.float32))

    idx8, c8 = _topk_call(imp2, ew2, rl_pad)
    idx64 = idx8[0, :TOP_K]
    c0 = c8[0, :TOP_K]
    c1 = c8[1, :TOP_K]

    out0, out1 = _route_call(sources.reshape(-1), idx64, c0, c1)
    return (out0, out1)


# speedup vs baseline: 2.3346x; 2.3346x over previous
import functools

import jax
import jax.numpy as jnp
from jax import lax
from jax.experimental import pallas as pl
from jax.experimental.pallas import tpu as pltpu
from jax.experimental.pallas import tpu_sc as plsc

N_SOURCES = 4096
TOP_K = 64
BATCH = 16384

_NC = 2
_NS = 16
_NW = _NC * _NS
_ROWS_PER_W = BATCH // _NW
_CHUNK = 128
_NCHUNKS = _ROWS_PER_W // _CHUNK


def _topk_body(imp_ref, ew_ref, rl_ref, idx_ref, c_ref):
    imp = imp_ref[:]
    gidx = (lax.broadcasted_iota(jnp.int32, (32, 128), 0) * 128
            + lax.broadcasted_iota(jnp.int32, (32, 128), 1))
    big = jnp.int32(1 << 30)
    neg = jnp.float32(-1e37)

    def sel_step(_, carry):
        impm, sel = carry
        m = jnp.max(impm)
        ij = jnp.min(jnp.where(impm == m, gidx, big))
        hit = gidx == ij
        return jnp.where(hit, neg, impm), sel | hit.astype(jnp.int32)

    _, sel = lax.fori_loop(
        0, TOP_K, sel_step, (imp, jnp.zeros((32, 128), jnp.int32)))

    lane = lax.broadcasted_iota(jnp.int32, (8, 128), 1)
    ew = ew_ref[:]

    def asc_step(k, carry):
        sel, idxv, ewv = carry
        ik = jnp.min(jnp.where(sel > 0, gidx, big))
        hit = gidx == ik
        ewk = jnp.sum(jnp.where(hit, ew, 0.0))
        put = lane == k
        return (sel * (1 - hit.astype(jnp.int32)),
                jnp.where(put, ik, idxv),
                jnp.where(put, ewk, ewv))

    _, idxv, ewv = lax.fori_loop(
        0, TOP_K, asc_step,
        (sel, jnp.zeros((8, 128), jnp.int32), jnp.zeros((8, 128), jnp.float32)))

    rl = rl_ref[:]
    m = jnp.max(rl, axis=1, keepdims=True)
    e = jnp.exp(rl - m)
    probs = e / jnp.sum(e, axis=1, keepdims=True)
    idx_ref[:] = idxv
    c_ref[:] = probs * ewv


def _topk_call(imp2, ew2, rl_pad):
    return pl.pallas_call(
        _topk_body,
        out_shape=[
            jax.ShapeDtypeStruct((8, 128), jnp.int32),
            jax.ShapeDtypeStruct((8, 128), jnp.float32),
        ],
    )(imp2, ew2, rl_pad)


def _route_body(src_hbm, idx_hbm, c0_hbm, c1_hbm, out0_hbm, out1_hbm,
                idx_v, c0_v, c1_v, idxl_a, idxl_b, data_a, data_b,
                out0_v, out1_v, sem_s, sem_a, sem_b):
    wid = lax.axis_index("s") * _NC + lax.axis_index("c")
    base = wid * _ROWS_PER_W

    pltpu.async_copy(idx_hbm, idx_v, sem_s).wait()
    pltpu.async_copy(c0_hbm, c0_v, sem_s).wait()
    pltpu.async_copy(c1_hbm, c1_v, sem_s).wait()

    idx_chunks = [
        (lambda v: v + (lax.shift_right_logical(v, 7) * 896))(
            idx_v[pl.ds(j * 16, 16)])
        for j in range(4)
    ]
    c0_chunks = [c0_v[pl.ds(j * 16, 16)] for j in range(4)]
    c1_chunks = [c1_v[pl.ds(j * 16, 16)] for j in range(4)]

    def build(t, idxl):
        row0 = base + t * _CHUNK

        def body(r, _):
            rg = row0 + r
            off = (lax.shift_right_logical(rg, 3) * 32768
                   + (rg & 7) * 128)
            for j in range(4):
                idxl[pl.ds(r * 64 + j * 16, 16)] = idx_chunks[j] + off
            return 0

        lax.fori_loop(0, _CHUNK, body, 0)

    lane16 = lax.iota(jnp.int32, 16)
    _gdn = lax.GatherDimensionNumbers(
        offset_dims=(), collapsed_slice_dims=(0,), start_index_map=(0,))

    def _rotsum(v):
        for sh in (8, 4, 2, 1):
            ridx = (lane16 + sh) & 15
            v = v + lax.gather(v, ridx[:, None], _gdn, (1,),
                               mode=lax.GatherScatterMode.PROMISE_IN_BOUNDS)
        return v

    def compute(t, data):
        def body(g, _):
            vec0 = jnp.zeros((16,), jnp.float32)
            vec1 = jnp.zeros((16,), jnp.float32)
            for rr in range(16):
                off = g * (16 * 64) + rr * 64
                acc0 = data[pl.ds(off, 16)] * c0_chunks[0]
                acc1 = data[pl.ds(off, 16)] * c1_chunks[0]
                for j in range(1, 4):
                    d = data[pl.ds(off + j * 16, 16)]
                    acc0 = acc0 + d * c0_chunks[j]
                    acc1 = acc1 + d * c1_chunks[j]
                put = lane16 == rr
                vec0 = jnp.where(put, _rotsum(acc0), vec0)
                vec1 = jnp.where(put, _rotsum(acc1), vec1)
            out0_v[pl.ds(t * _CHUNK + g * 16, 16)] = vec0
            out1_v[pl.ds(t * _CHUNK + g * 16, 16)] = vec1
            return 0

        lax.fori_loop(0, _CHUNK // 16, body, 0)

    bufs = [(idxl_a, data_a, sem_a), (idxl_b, data_b, sem_b)]

    build(0, bufs[0][0])
    copies = {0: pltpu.async_copy(src_hbm.at[bufs[0][0]], bufs[0][1], bufs[0][2])}
    for t in range(_NCHUNKS):
        if t + 1 < _NCHUNKS:
            nb = bufs[(t + 1) % 2]
            build(t + 1, nb[0])
            copies[t + 1] = pltpu.async_copy(src_hbm.at[nb[0]], nb[1], nb[2])
        copies[t].wait()
        compute(t, bufs[t % 2][1])

    pltpu.async_copy(out0_v, out0_hbm.at[pl.ds(base, _ROWS_PER_W)], sem_s).wait()
    pltpu.async_copy(out1_v, out1_hbm.at[pl.ds(base, _ROWS_PER_W)], sem_s).wait()


@functools.partial(jax.jit, static_argnums=())
def _route_call(src_flat, idx64, c0, c1):
    mesh = plsc.VectorSubcoreMesh(core_axis_name="c", subcore_axis_name="s")
    f = pl.kernel(
        _route_body,
        out_type=[
            jax.ShapeDtypeStruct((BATCH,), jnp.float32),
            jax.ShapeDtypeStruct((BATCH,), jnp.float32),
        ],
        mesh=mesh,
        scratch_types=[
            pltpu.VMEM((TOP_K,), jnp.int32),
            pltpu.VMEM((TOP_K,), jnp.float32),
            pltpu.VMEM((TOP_K,), jnp.float32),
            pltpu.VMEM((_CHUNK * 64,), jnp.int32),
            pltpu.VMEM((_CHUNK * 64,), jnp.int32),
            pltpu.VMEM((_CHUNK * 64,), jnp.float32),
            pltpu.VMEM((_CHUNK * 64,), jnp.float32),
            pltpu.VMEM((_ROWS_PER_W,), jnp.float32),
            pltpu.VMEM((_ROWS_PER_W,), jnp.float32),
            pltpu.SemaphoreType.DMA,
            pltpu.SemaphoreType.DMA,
            pltpu.SemaphoreType.DMA,
        ],
    )
    return f(src_flat, idx64, c0, c1)


def kernel(sources, importance_logits, edge_weights, route_logits):
    imp2 = importance_logits.reshape(32, 128)
    ew2 = edge_weights.reshape(32, 128)
    rl_pad = jnp.full((8, 128), -1e30, jnp.float32)
    rl_pad = rl_pad.at[:2, :TOP_K].set(route_logits.astype(jnp.float32))

    idx8, c8 = _topk_call(imp2, ew2, rl_pad)
    idx64 = idx8[0, :TOP_K]
    c0 = c8[0, :TOP_K]
    c1 = c8[1, :TOP_K]

    src_tiled = sources.reshape(2048, 8, 32, 128).transpose(0, 2, 1, 3)
    out0, out1 = _route_call(src_tiled.reshape(-1), idx64, c0, c1)
    return (out0, out1)


# speedup vs baseline: 3.2338x; 1.3851x over previous
import functools

import jax
import jax.numpy as jnp
from jax import lax
from jax.experimental import pallas as pl
from jax.experimental.pallas import tpu as pltpu
from jax.experimental.pallas import tpu_sc as plsc

N_SOURCES = 4096
TOP_K = 64
BATCH = 16384

_NC = 2
_NS = 16
_NW = _NC * _NS
_ROWS_PER_W = BATCH // _NW
_CHUNK = 128
_NCHUNKS = _ROWS_PER_W // _CHUNK


def _topk_body(imp_ref, rl_ref, sel_ref, probs_ref):
    imp = imp_ref[:]
    fbits = lax.bitcast_convert_type(imp, jnp.int32)
    skey = jnp.where(fbits >= 0, fbits, fbits ^ jnp.int32(0x7FFFFFFF))
    gidx = (lax.broadcasted_iota(jnp.int32, (32, 128), 0) * 128
            + lax.broadcasted_iota(jnp.int32, (32, 128), 1))
    sign = jnp.int32(-2147483648)

    def bit_step(i, tu):
        cand = tu | (jnp.int32(1) << (31 - i))
        thr = cand ^ sign
        cnt = jnp.sum((skey >= thr).astype(jnp.int32))
        return jnp.where(cnt >= TOP_K, cand, tu)

    tu = lax.fori_loop(0, 32, bit_step, jnp.int32(0))
    kb = tu ^ sign
    m = jnp.sum((skey > kb).astype(jnp.int32))
    r = TOP_K - m

    def tie_step(i, ti):
        cand = ti | (jnp.int32(1) << (12 - i))
        cnt = jnp.sum(((skey == kb) & (gidx < cand)).astype(jnp.int32))
        return jnp.where(cnt <= r, cand, ti)

    ti = lax.fori_loop(0, 13, tie_step, jnp.int32(0))
    sel = (skey > kb) | ((skey == kb) & (gidx < ti))
    sel_ref[:] = sel.astype(jnp.int32)

    rl = rl_ref[:]
    mx = jnp.max(rl, axis=1, keepdims=True)
    e = jnp.exp(rl - mx)
    probs_ref[:] = e / jnp.sum(e, axis=1, keepdims=True)


def _topk_call(imp2, rl_pad):
    return pl.pallas_call(
        _topk_body,
        out_shape=[
            jax.ShapeDtypeStruct((32, 128), jnp.int32),
            jax.ShapeDtypeStruct((8, 128), jnp.float32),
        ],
    )(imp2, rl_pad)


def _route_body(src_hbm, selm_hbm, p0_hbm, p1_hbm, ew_hbm, out0_hbm, out1_hbm,
                selm_v, ew_v, p0_v, p1_v, idx_buf, ewsel_buf,
                idxl_a, idxl_b, data_a, data_b,
                out0_v, out1_v, sem_s, sem_a, sem_b):
    wid = lax.axis_index("s") * _NC + lax.axis_index("c")
    base = wid * _ROWS_PER_W

    pltpu.async_copy(selm_hbm, selm_v, sem_s).wait()
    pltpu.async_copy(ew_hbm, ew_v, sem_s).wait()
    pltpu.async_copy(p0_hbm, p0_v, sem_s).wait()
    pltpu.async_copy(p1_hbm, p1_v, sem_s).wait()

    lane16 = lax.iota(jnp.int32, 16)
    _gdn = lax.GatherDimensionNumbers(
        offset_dims=(), collapsed_slice_dims=(0,), start_index_map=(0,))

    def _gat(v, idx):
        return lax.gather(v, idx[:, None], _gdn, (1,),
                          mode=lax.GatherScatterMode.PROMISE_IN_BOUNDS)

    def comp_step(c, offv):
        mvec = selm_v[pl.ds(c * 16, 16)]
        mb = mvec > 0
        pre = mvec
        for sh in (1, 2, 4, 8):
            shifted = _gat(pre, (lane16 - sh) & 15)
            pre = pre + jnp.where(lane16 >= sh, shifted,
                                  jnp.zeros((16,), jnp.int32))
        posv = offv + pre - 1
        idxvec = c * 16 + lane16
        plsc.store_scatter(idx_buf, [posv], idxvec, mask=mb)
        plsc.store_scatter(ewsel_buf, [posv], ew_v[pl.ds(c * 16, 16)], mask=mb)
        return offv + _gat(pre, jnp.full((16,), 15, jnp.int32))

    lax.fori_loop(0, N_SOURCES // 16, comp_step,
                  jnp.zeros((16,), jnp.int32))

    idx_chunks = [
        (lambda v: v + (lax.shift_right_logical(v, 7) * 896))(
            idx_buf[pl.ds(j * 16, 16)])
        for j in range(4)
    ]
    c0_chunks = [p0_v[pl.ds(j * 16, 16)] * ewsel_buf[pl.ds(j * 16, 16)]
                 for j in range(4)]
    c1_chunks = [p1_v[pl.ds(j * 16, 16)] * ewsel_buf[pl.ds(j * 16, 16)]
                 for j in range(4)]

    def build(t, idxl):
        row0 = base + t * _CHUNK

        def body(r, _):
            rg = row0 + r
            off = (lax.shift_right_logical(rg, 3) * 32768
                   + (rg & 7) * 128)
            for j in range(4):
                idxl[pl.ds(r * 64 + j * 16, 16)] = idx_chunks[j] + off
            return 0

        lax.fori_loop(0, _CHUNK, body, 0)

    def _rotsum(v):
        for sh in (8, 4, 2, 1):
            v = v + _gat(v, (lane16 + sh) & 15)
        return v

    def compute(t, data):
        def body(g, _):
            vec0 = jnp.zeros((16,), jnp.float32)
            vec1 = jnp.zeros((16,), jnp.float32)
            for rr in range(16):
                off = g * (16 * 64) + rr * 64
                acc0 = data[pl.ds(off, 16)] * c0_chunks[0]
                acc1 = data[pl.ds(off, 16)] * c1_chunks[0]
                for j in range(1, 4):
                    d = data[pl.ds(off + j * 16, 16)]
                    acc0 = acc0 + d * c0_chunks[j]
                    acc1 = acc1 + d * c1_chunks[j]
                put = lane16 == rr
                vec0 = jnp.where(put, _rotsum(acc0), vec0)
                vec1 = jnp.where(put, _rotsum(acc1), vec1)
            out0_v[pl.ds(t * _CHUNK + g * 16, 16)] = vec0
            out1_v[pl.ds(t * _CHUNK + g * 16, 16)] = vec1
            return 0

        lax.fori_loop(0, _CHUNK // 16, body, 0)

    bufs = [(idxl_a, data_a, sem_a), (idxl_b, data_b, sem_b)]

    build(0, bufs[0][0])
    copies = {0: pltpu.async_copy(src_hbm.at[bufs[0][0]], bufs[0][1], bufs[0][2])}
    for t in range(_NCHUNKS):
        if t + 1 < _NCHUNKS:
            nb = bufs[(t + 1) % 2]
            build(t + 1, nb[0])
            copies[t + 1] = pltpu.async_copy(src_hbm.at[nb[0]], nb[1], nb[2])
        copies[t].wait()
        compute(t, bufs[t % 2][1])

    pltpu.async_copy(out0_v, out0_hbm.at[pl.ds(base, _ROWS_PER_W)], sem_s).wait()
    pltpu.async_copy(out1_v, out1_hbm.at[pl.ds(base, _ROWS_PER_W)], sem_s).wait()


@functools.partial(jax.jit, static_argnums=())
def _route_call(src_flat, selflat, p0, p1, ew):
    mesh = plsc.VectorSubcoreMesh(core_axis_name="c", subcore_axis_name="s")
    f = pl.kernel(
        _route_body,
        out_type=[
            jax.ShapeDtypeStruct((BATCH,), jnp.float32),
            jax.ShapeDtypeStruct((BATCH,), jnp.float32),
        ],
        mesh=mesh,
        compiler_params=pltpu.CompilerParams(needs_layout_passes=False),
        scratch_types=[
            pltpu.VMEM((N_SOURCES,), jnp.int32),
            pltpu.VMEM((N_SOURCES,), jnp.float32),
            pltpu.VMEM((TOP_K,), jnp.float32),
            pltpu.VMEM((TOP_K,), jnp.float32),
            pltpu.VMEM((80,), jnp.int32),
            pltpu.VMEM((80,), jnp.float32),
            pltpu.VMEM((_CHUNK * 64,), jnp.int32),
            pltpu.VMEM((_CHUNK * 64,), jnp.int32),
            pltpu.VMEM((_CHUNK * 64,), jnp.float32),
            pltpu.VMEM((_CHUNK * 64,), jnp.float32),
            pltpu.VMEM((_ROWS_PER_W,), jnp.float32),
            pltpu.VMEM((_ROWS_PER_W,), jnp.float32),
            pltpu.SemaphoreType.DMA,
            pltpu.SemaphoreType.DMA,
            pltpu.SemaphoreType.DMA,
        ],
    )
    return f(src_flat, selflat, p0, p1, ew)


def kernel(sources, importance_logits, edge_weights, route_logits):
    imp2 = importance_logits.reshape(32, 128)
    rl_pad = jnp.full((8, 128), -1e30, jnp.float32)
    rl_pad = rl_pad.at[:2, :TOP_K].set(route_logits.astype(jnp.float32))

    sel2, probs8 = _topk_call(imp2, rl_pad)
    p0 = probs8[0, :TOP_K]
    p1 = probs8[1, :TOP_K]

    src_tiled = sources.reshape(2048, 8, 32, 128).transpose(0, 2, 1, 3)
    out0, out1 = _route_call(src_tiled.reshape(-1), sel2.reshape(-1),
                             p0, p1, edge_weights)
    return (out0, out1)
